# Initial kernel scaffold; baseline (speedup 1.0000x reference)
import jax
import jax.numpy as jnp
from jax.experimental import pallas as pl


def kernel(hidden_states, router_weight, ws, w2s):
    raise NotImplementedError("write your pallas kernel here")


# speedup vs baseline: 1.4151x; 1.4151x over previous
import jax
import jax.numpy as jnp
from jax.experimental import pallas as pl
from jax.experimental.pallas import tpu as pltpu

D_MODEL = 1024
N_EXPERTS = 8
TOP_K = 2
FFN = 4096
T = 2048

BF = 512
N_F = FFN // BF


def _router_kernel(x_ref, rw_ref, combine_ref):
    x = x_ref[...]
    rw = rw_ref[...]
    logits = jax.lax.dot_general(
        x, rw, (((1,), (1,)), ((), ())), preferred_element_type=jnp.float32
    )
    m = jnp.max(logits, axis=1, keepdims=True)
    ex = jnp.exp(logits - m)
    probs = ex / jnp.sum(ex, axis=1, keepdims=True)
    idx = jax.lax.broadcasted_iota(jnp.int32, probs.shape, 1)
    big = jnp.int32(N_EXPERTS + 1)
    p1 = jnp.max(probs, axis=1, keepdims=True)
    i1 = jnp.min(jnp.where(probs == p1, idx, big), axis=1, keepdims=True)
    m1 = idx == i1
    probs2 = jnp.where(m1, -1.0, probs)
    p2 = jnp.max(probs2, axis=1, keepdims=True)
    i2 = jnp.min(jnp.where(probs2 == p2, idx, big), axis=1, keepdims=True)
    m2 = idx == i2
    denom = p1 + p2
    combine = (jnp.where(m1, probs, 0.0) + jnp.where(m2, probs, 0.0)) / denom
    combine_ref[...] = combine.T


def _expert_kernel(x_ref, comb_ref, w1_ref, v1_ref, w2_ref, out_ref):
    e = pl.program_id(0)
    f = pl.program_id(1)

    @pl.when((e == 0) & (f == 0))
    def _init():
        out_ref[...] = jnp.zeros_like(out_ref)

    x = x_ref[...]
    gate = jax.lax.dot_general(
        x, w1_ref[0], (((1,), (1,)), ((), ())), preferred_element_type=jnp.float32
    )
    up = jax.lax.dot_general(
        x, v1_ref[0], (((1,), (1,)), ((), ())), preferred_element_type=jnp.float32
    )
    act = gate * jax.lax.logistic(gate) * up
    eoh = (jax.lax.broadcasted_iota(jnp.int32, (N_EXPERTS, 1), 0) == e).astype(
        jnp.float32
    )
    c = jax.lax.dot_general(
        comb_ref[...], eoh, (((0,), (0,)), ((), ())), preferred_element_type=jnp.float32
    )
    act = act * c
    out_ref[...] += jax.lax.dot_general(
        act, w2_ref[0], (((1,), (1,)), ((), ())), preferred_element_type=jnp.float32
    )


def kernel(hidden_states, router_weight, ws, w2s):
    x = hidden_states.reshape(-1, D_MODEL)

    combine = pl.pallas_call(
        _router_kernel,
        out_shape=jax.ShapeDtypeStruct((N_EXPERTS, T), jnp.float32),
    )(x, router_weight)

    out = pl.pallas_call(
        _expert_kernel,
        grid=(N_EXPERTS, N_F),
        in_specs=[
            pl.BlockSpec((T, D_MODEL), lambda e, f: (0, 0)),
            pl.BlockSpec((N_EXPERTS, T), lambda e, f: (0, 0)),
            pl.BlockSpec((1, BF, D_MODEL), lambda e, f: (e, f, 0)),
            pl.BlockSpec((1, BF, D_MODEL), lambda e, f: (e, f + N_F, 0)),
            pl.BlockSpec((1, D_MODEL, BF), lambda e, f: (e, 0, f)),
        ],
        out_specs=pl.BlockSpec((T, D_MODEL), lambda e, f: (0, 0)),
        out_shape=jax.ShapeDtypeStruct((T, D_MODEL), jnp.float32),
        compiler_params=pltpu.CompilerParams(
            dimension_semantics=("arbitrary", "arbitrary"),
        ),
    )(x, combine, ws, ws, w2s)

    return out.reshape(hidden_states.shape)
